# baseline (speedup 1.0000x reference)
import dataclasses
import functools

import jax
import jax.numpy as jnp
from jax import lax
from jax.experimental import pallas as pl
from jax.experimental.pallas import tpu as pltpu
from jax.experimental.pallas import tpu_sc as plsc

N = 10000
E = 160000
NSUB = 16
SLAB = 624
LAST_SLAB = N - (NSUB - 1) * SLAB

_MESH = plsc.VectorSubcoreMesh(core_axis_name="c", subcore_axis_name="s")
_f32 = jnp.float32
_i32 = jnp.int32

_SC_PARAMS = pltpu.CompilerParams()
if "needs_layout_passes" in pltpu.CompilerParams.__dataclass_fields__:
    _SC_PARAMS = dataclasses.replace(_SC_PARAMS, needs_layout_passes=False)


def _slab_copy(s, do_copy):
    base = pl.multiple_of(s * SLAB, 8)

    @pl.when(s < NSUB - 1)
    def _():
        do_copy(base, SLAB)

    @pl.when(s == NSUB - 1)
    def _():
        do_copy((NSUB - 1) * SLAB, LAST_SLAB)


DG_CH = 40
DG_NCH = E // (2 * NSUB) // DG_CH
DG_ESUB = E // (2 * NSUB)


def _deg_body(col3d_hbm, ew_hbm, zeros_hbm, out_a, out_b,
              acc, col2d, ewf, v0, v1, sv0, sv1):
    c = lax.axis_index("c")
    s = lax.axis_index("s")
    tid = c * NSUB + s
    _slab_copy(s, lambda b, n: pltpu.sync_copy(zeros_hbm.at[pl.ds(b, n)],
                                               acc.at[pl.ds(b, n)]))
    pltpu.sync_copy(col3d_hbm.at[tid], col2d)
    pltpu.sync_copy(ew_hbm.at[pl.ds(tid * DG_ESUB, DG_ESUB)], ewf)
    plsc.subcore_barrier()

    def build(m, vb):
        ebase = m * DG_CH

        @plsc.parallel_loop(0, DG_CH, unroll=4)
        def _edge(e):
            w = plsc.load_gather(ewf, [jnp.full((16,), ebase + e, _i32)])
            vb[e, pl.ds(0, 16)] = w

    def scat(m, vb, sem):
        pltpu.async_copy(vb, acc.at[col2d.at[m]], sem, add=True)

    def wait_scat(vb, sem):
        pltpu.make_async_copy(zeros_hbm.at[pl.ds(0, DG_CH)], vb, sem).wait()

    build(0, v0)
    scat(0, v0, sv0)

    @pl.loop(0, (DG_NCH - 1) // 2)
    def _pair(i):
        m0 = 2 * i + 1
        build(m0, v1)
        scat(m0, v1, sv1)
        wait_scat(v0, sv0)
        build(m0 + 1, v0)
        scat(m0 + 1, v0, sv0)
        wait_scat(v1, sv1)

    wait_scat(v0, sv0)
    plsc.subcore_barrier()

    @pl.when(c == 0)
    def _():
        _slab_copy(s, lambda b, n: pltpu.sync_copy(acc.at[pl.ds(b, n)],
                                                   out_a.at[pl.ds(b, n)]))

    @pl.when(c == 1)
    def _():
        _slab_copy(s, lambda b, n: pltpu.sync_copy(acc.at[pl.ds(b, n)],
                                                   out_b.at[pl.ds(b, n)]))


_deg_call = pl.kernel(
    _deg_body,
    out_type=(jax.ShapeDtypeStruct((N, 128), _f32),
              jax.ShapeDtypeStruct((N, 128), _f32)),
    mesh=_MESH,
    compiler_params=_SC_PARAMS,
    scratch_types=[
        pltpu.VMEM_SHARED((N, 128), _f32),
        pltpu.VMEM((DG_NCH, DG_CH), _i32),
        pltpu.VMEM((DG_ESUB,), _f32),
        pltpu.VMEM((DG_CH, 128), _f32),
        pltpu.VMEM((DG_CH, 128), _f32),
        pltpu.SemaphoreType.DMA,
        pltpu.SemaphoreType.DMA,
    ],
)


def _agg_chunks(g_ref, acc, rowf, col2d, ew_hbm, ew_base,
                ew0, ew1, g0, g1, sem0, sem1, seme0, seme1,
                n_chunks, ch):
    def gather(m, gb, sem):
        off = pl.multiple_of(m * ch, 8)
        pltpu.async_copy(g_ref.at[rowf.at[pl.ds(off, ch)]], gb, sem)

    def fetch_ew(m, eb, sem):
        off = pl.multiple_of(ew_base + m * ch, 8)
        pltpu.async_copy(ew_hbm.at[pl.ds(off, ch)], eb, sem)

    def mult_scatter(m, gb, sem, eb, seme):
        pltpu.make_async_copy(g_ref.at[pl.ds(0, ch)], gb, sem).wait()
        pltpu.make_async_copy(ew_hbm.at[pl.ds(0, ch)], eb, seme).wait()

        @plsc.parallel_loop(0, ch, unroll=8)
        def _edge(e):
            w = plsc.load_gather(eb, [jnp.full((16,), e, _i32)])
            for f in range(8):
                sl = pl.ds(f * 16, 16)
                gb[e, sl] = gb[e, sl] * w

        pltpu.sync_copy(gb, acc.at[col2d.at[m]], add=True)

    fetch_ew(0, ew0, seme0)
    gather(0, g0, sem0)

    @pl.loop(0, (n_chunks - 1) // 2)
    def _pair(i):
        m0 = i * 2
        fetch_ew(m0 + 1, ew1, seme1)
        gather(m0 + 1, g1, sem1)
        mult_scatter(m0, g0, sem0, ew0, seme0)
        fetch_ew(m0 + 2, ew0, seme0)
        gather(m0 + 2, g0, sem0)
        mult_scatter(m0 + 1, g1, sem1, ew1, seme1)

    mult_scatter(n_chunks - 1, g0, sem0, ew0, seme0)


A1_CH = 80
A1_NCH = E // NSUB // A1_CH
A1_ESUB = E // NSUB


def _agg1_body(g_a, g_b, row_hbm, col2d_hbm, ew_hbm, out_a, out_b,
               acc, rowf, col2d, ew0, ew1, g0, g1,
               sem0, sem1, seme0, seme1):
    c = lax.axis_index("c")
    s = lax.axis_index("s")

    def phase(g_ref, out_ref):
        _slab_copy(s, lambda b, n: pltpu.sync_copy(g_ref.at[pl.ds(b, n)],
                                                   acc.at[pl.ds(b, n)]))
        pltpu.sync_copy(row_hbm.at[pl.ds(s * A1_ESUB, A1_ESUB)], rowf)
        pltpu.sync_copy(col2d_hbm.at[s], col2d)
        plsc.subcore_barrier()

        _agg_chunks(g_ref, acc, rowf, col2d, ew_hbm, s * A1_ESUB,
                    ew0, ew1, g0, g1, sem0, sem1, seme0, seme1,
                    A1_NCH, A1_CH)

        plsc.subcore_barrier()
        _slab_copy(s, lambda b, n: pltpu.sync_copy(acc.at[pl.ds(b, n)],
                                                   out_ref.at[pl.ds(b, n)]))

    @pl.when(c == 0)
    def _():
        phase(g_a, out_a)

    @pl.when(c == 1)
    def _():
        phase(g_b, out_b)


_agg1_call = pl.kernel(
    _agg1_body,
    out_type=(jax.ShapeDtypeStruct((N, 128), _f32),
              jax.ShapeDtypeStruct((N, 128), _f32)),
    mesh=_MESH,
    compiler_params=_SC_PARAMS,
    scratch_types=[
        pltpu.VMEM_SHARED((N, 128), _f32),
        pltpu.VMEM((A1_ESUB,), _i32),
        pltpu.VMEM((A1_NCH, A1_CH), _i32),
        pltpu.VMEM((A1_CH,), _f32),
        pltpu.VMEM((A1_CH,), _f32),
        pltpu.VMEM((A1_CH, 128), _f32),
        pltpu.VMEM((A1_CH, 128), _f32),
        pltpu.SemaphoreType.DMA,
        pltpu.SemaphoreType.DMA,
        pltpu.SemaphoreType.DMA,
        pltpu.SemaphoreType.DMA,
    ],
)


A2_CH = 40
A2_NCH = E // (2 * NSUB) // A2_CH
A2_ESUB = E // (2 * NSUB)


def _agg2_body(g_hbm, zeros_hbm, row_hbm, col2d_hbm, ew_hbm, out_a, out_b,
               acc, rowf, col2d, ew0, ew1, g0, g1,
               sem0, sem1, seme0, seme1):
    c = lax.axis_index("c")
    s = lax.axis_index("s")
    tid = c * NSUB + s

    def init_from(src):
        _slab_copy(s, lambda b, n: pltpu.sync_copy(src.at[pl.ds(b, n)],
                                                   acc.at[pl.ds(b, n)]))

    @pl.when(c == 0)
    def _():
        init_from(g_hbm)

    @pl.when(c == 1)
    def _():
        init_from(zeros_hbm)

    pltpu.sync_copy(row_hbm.at[pl.ds(tid * A2_ESUB, A2_ESUB)], rowf)
    pltpu.sync_copy(col2d_hbm.at[tid], col2d)
    plsc.subcore_barrier()

    _agg_chunks(g_hbm, acc, rowf, col2d, ew_hbm, tid * A2_ESUB,
                ew0, ew1, g0, g1, sem0, sem1, seme0, seme1,
                A2_NCH, A2_CH)

    plsc.subcore_barrier()

    @pl.when(c == 0)
    def _():
        _slab_copy(s, lambda b, n: pltpu.sync_copy(acc.at[pl.ds(b, n)],
                                                   out_a.at[pl.ds(b, n)]))

    @pl.when(c == 1)
    def _():
        _slab_copy(s, lambda b, n: pltpu.sync_copy(acc.at[pl.ds(b, n)],
                                                   out_b.at[pl.ds(b, n)]))


_agg2_call = pl.kernel(
    _agg2_body,
    out_type=(jax.ShapeDtypeStruct((N, 128), _f32),
              jax.ShapeDtypeStruct((N, 128), _f32)),
    mesh=_MESH,
    compiler_params=_SC_PARAMS,
    scratch_types=[
        pltpu.VMEM_SHARED((N, 128), _f32),
        pltpu.VMEM((A2_ESUB,), _i32),
        pltpu.VMEM((A2_NCH, A2_CH), _i32),
        pltpu.VMEM((A2_CH,), _f32),
        pltpu.VMEM((A2_CH,), _f32),
        pltpu.VMEM((A2_CH, 128), _f32),
        pltpu.VMEM((A2_CH, 128), _f32),
        pltpu.SemaphoreType.DMA,
        pltpu.SemaphoreType.DMA,
        pltpu.SemaphoreType.DMA,
        pltpu.SemaphoreType.DMA,
    ],
)


def _mm1_body(x_ref, w_ref, h_ref):
    h_ref[...] = jnp.dot(x_ref[...], w_ref[...],
                         preferred_element_type=_f32)


_mm1_call = pl.pallas_call(
    _mm1_body,
    out_shape=jax.ShapeDtypeStruct((N, 256), _f32),
)


def _dis(da_ref, db_ref):
    deg = da_ref[:, 0:1] + db_ref[:, 0:1] + 1.0
    return lax.rsqrt(deg)


def _scale1_body(h_ref, da_ref, db_ref, ga_ref, gb_ref):
    g = h_ref[...] * _dis(da_ref, db_ref)
    ga_ref[...] = g[:, :128]
    gb_ref[...] = g[:, 128:]


_scale1_call = pl.pallas_call(
    _scale1_body,
    out_shape=(jax.ShapeDtypeStruct((N, 128), _f32),
               jax.ShapeDtypeStruct((N, 128), _f32)),
)


def _mm2_body(aa_ref, ab_ref, da_ref, db_ref, w2_ref, g2_ref):
    dis = _dis(da_ref, db_ref)
    out1 = jnp.concatenate([aa_ref[...], ab_ref[...]], axis=1) * dis
    out1 = jnp.maximum(out1, 0.0)
    g2_ref[...] = jnp.dot(out1, w2_ref[...], preferred_element_type=_f32) * dis


_mm2_call = pl.pallas_call(
    _mm2_body,
    out_shape=jax.ShapeDtypeStruct((N, 128), _f32),
)


def _fin_body(ba_ref, bb_ref, da_ref, db_ref, out_ref):
    dis = _dis(da_ref, db_ref)
    out_ref[...] = (ba_ref[...] + bb_ref[...]) * dis


_fin_call = pl.pallas_call(
    _fin_body,
    out_shape=jax.ShapeDtypeStruct((N, 128), _f32),
)


def kernel(x, edge_index, edge_weight, W1, W2):
    row = edge_index[0].astype(_i32)
    col = edge_index[1].astype(_i32)
    ew = edge_weight.astype(_f32)
    zeros128 = jnp.zeros((N, 128), _f32)
    col2d_a1 = col.reshape(NSUB, A1_NCH, A1_CH)
    col2d_a2 = col.reshape(2 * NSUB, A2_NCH, A2_CH)

    col3d_dg = col.reshape(2 * NSUB, DG_NCH, DG_CH)
    da, db = _deg_call(col3d_dg, ew, zeros128)
    h1 = _mm1_call(x, W1)
    ga, gb = _scale1_call(h1, da, db)
    aa, ab = _agg1_call(ga, gb, row, col2d_a1, ew)
    g2 = _mm2_call(aa, ab, da, db, W2)
    ba, bb = _agg2_call(g2, zeros128, row, col2d_a2, ew)
    return _fin_call(ba, bb, da, db)


# speedup vs baseline: 1.0124x; 1.0124x over previous
import dataclasses
import functools

import jax
import jax.numpy as jnp
from jax import lax
from jax.experimental import pallas as pl
from jax.experimental.pallas import tpu as pltpu
from jax.experimental.pallas import tpu_sc as plsc

N = 10000
E = 160000
NSUB = 16
SLAB = 624
LAST_SLAB = N - (NSUB - 1) * SLAB

_MESH = plsc.VectorSubcoreMesh(core_axis_name="c", subcore_axis_name="s")
_f32 = jnp.float32
_i32 = jnp.int32

_SC_PARAMS = pltpu.CompilerParams()
if "needs_layout_passes" in pltpu.CompilerParams.__dataclass_fields__:
    _SC_PARAMS = dataclasses.replace(_SC_PARAMS, needs_layout_passes=False)


def _slab_copy(s, do_copy):
    base = pl.multiple_of(s * SLAB, 8)

    @pl.when(s < NSUB - 1)
    def _():
        do_copy(base, SLAB)

    @pl.when(s == NSUB - 1)
    def _():
        do_copy((NSUB - 1) * SLAB, LAST_SLAB)


DG_CH = 40
DG_NCH = E // (2 * NSUB) // DG_CH
DG_ESUB = E // (2 * NSUB)


def _deg_body(col3d_hbm, ew_hbm, zeros_hbm, out_a, out_b,
              acc, col2d, ewf, v0, v1, sv0, sv1):
    c = lax.axis_index("c")
    s = lax.axis_index("s")
    tid = c * NSUB + s

    @pl.loop(0, DG_CH)
    def _zrow(r):
        for f in range(8):
            v0[r, pl.ds(f * 16, 16)] = jnp.zeros((16,), _f32)

    base = pl.multiple_of(s * SLAB, 8)
    for k in range(SLAB // DG_CH):
        pltpu.sync_copy(v0, acc.at[pl.ds(base + k * DG_CH, DG_CH)])
    pltpu.sync_copy(v0.at[pl.ds(0, SLAB - (SLAB // DG_CH) * DG_CH)],
                    acc.at[pl.ds(base + (SLAB // DG_CH) * DG_CH,
                                 SLAB - (SLAB // DG_CH) * DG_CH)])

    @pl.when(s == NSUB - 1)
    def _():
        extra = LAST_SLAB - SLAB
        pltpu.sync_copy(v0.at[pl.ds(0, extra)],
                        acc.at[pl.ds(N - extra, extra)])

    pltpu.sync_copy(col3d_hbm.at[tid], col2d)
    pltpu.sync_copy(ew_hbm.at[pl.ds(tid * DG_ESUB, DG_ESUB)], ewf)
    plsc.subcore_barrier()

    def build(m, vb):
        ebase = m * DG_CH

        @plsc.parallel_loop(0, DG_CH, unroll=4)
        def _edge(e):
            w = plsc.load_gather(ewf, [jnp.full((16,), ebase + e, _i32)])
            vb[e, pl.ds(0, 16)] = w

    def scat(m, vb, sem):
        pltpu.async_copy(vb, acc.at[col2d.at[m]], sem, add=True)

    def wait_scat(vb, sem):
        pltpu.make_async_copy(zeros_hbm.at[pl.ds(0, DG_CH)], vb, sem).wait()

    build(0, v0)
    scat(0, v0, sv0)

    @pl.loop(0, (DG_NCH - 1) // 2)
    def _pair(i):
        m0 = 2 * i + 1
        build(m0, v1)
        scat(m0, v1, sv1)
        wait_scat(v0, sv0)
        build(m0 + 1, v0)
        scat(m0 + 1, v0, sv0)
        wait_scat(v1, sv1)

    wait_scat(v0, sv0)
    plsc.subcore_barrier()

    @pl.when(c == 0)
    def _():
        _slab_copy(s, lambda b, n: pltpu.sync_copy(acc.at[pl.ds(b, n)],
                                                   out_a.at[pl.ds(b, n)]))

    @pl.when(c == 1)
    def _():
        _slab_copy(s, lambda b, n: pltpu.sync_copy(acc.at[pl.ds(b, n)],
                                                   out_b.at[pl.ds(b, n)]))


_deg_call = pl.kernel(
    _deg_body,
    out_type=(jax.ShapeDtypeStruct((N, 128), _f32),
              jax.ShapeDtypeStruct((N, 128), _f32)),
    mesh=_MESH,
    compiler_params=_SC_PARAMS,
    scratch_types=[
        pltpu.VMEM_SHARED((N, 128), _f32),
        pltpu.VMEM((DG_NCH, DG_CH), _i32),
        pltpu.VMEM((DG_ESUB,), _f32),
        pltpu.VMEM((DG_CH, 128), _f32),
        pltpu.VMEM((DG_CH, 128), _f32),
        pltpu.SemaphoreType.DMA,
        pltpu.SemaphoreType.DMA,
    ],
)


def _agg_chunks(g_ref, acc, rowf, col2d, ew_hbm, ew_base,
                ew0, ew1, g0, g1, sem0, sem1, seme0, seme1,
                n_chunks, ch):
    def gather(m, gb, sem):
        off = pl.multiple_of(m * ch, 8)
        pltpu.async_copy(g_ref.at[rowf.at[pl.ds(off, ch)]], gb, sem)

    def fetch_ew(m, eb, sem):
        off = pl.multiple_of(ew_base + m * ch, 8)
        pltpu.async_copy(ew_hbm.at[pl.ds(off, ch)], eb, sem)

    def mult_scatter(m, gb, sem, eb, seme):
        pltpu.make_async_copy(g_ref.at[pl.ds(0, ch)], gb, sem).wait()
        pltpu.make_async_copy(ew_hbm.at[pl.ds(0, ch)], eb, seme).wait()

        @plsc.parallel_loop(0, ch, unroll=4)
        def _edge(e):
            w = plsc.load_gather(eb, [jnp.full((16,), e, _i32)])
            for f in range(8):
                sl = pl.ds(f * 16, 16)
                gb[e, sl] = gb[e, sl] * w

        pltpu.sync_copy(gb, acc.at[col2d.at[m]], add=True)

    fetch_ew(0, ew0, seme0)
    gather(0, g0, sem0)

    @pl.loop(0, (n_chunks - 1) // 2)
    def _pair(i):
        m0 = i * 2
        fetch_ew(m0 + 1, ew1, seme1)
        gather(m0 + 1, g1, sem1)
        mult_scatter(m0, g0, sem0, ew0, seme0)
        fetch_ew(m0 + 2, ew0, seme0)
        gather(m0 + 2, g0, sem0)
        mult_scatter(m0 + 1, g1, sem1, ew1, seme1)

    mult_scatter(n_chunks - 1, g0, sem0, ew0, seme0)


A1_CH = 80
A1_NCH = E // NSUB // A1_CH
A1_ESUB = E // NSUB


def _agg1_body(g_a, g_b, row_hbm, col2d_hbm, ew_hbm, out_a, out_b,
               acc, rowf, col2d, ew0, ew1, g0, g1,
               sem0, sem1, seme0, seme1):
    c = lax.axis_index("c")
    s = lax.axis_index("s")

    def phase(g_ref, out_ref):
        _slab_copy(s, lambda b, n: pltpu.sync_copy(g_ref.at[pl.ds(b, n)],
                                                   acc.at[pl.ds(b, n)]))
        pltpu.sync_copy(row_hbm.at[pl.ds(s * A1_ESUB, A1_ESUB)], rowf)
        pltpu.sync_copy(col2d_hbm.at[s], col2d)
        plsc.subcore_barrier()

        _agg_chunks(g_ref, acc, rowf, col2d, ew_hbm, s * A1_ESUB,
                    ew0, ew1, g0, g1, sem0, sem1, seme0, seme1,
                    A1_NCH, A1_CH)

        plsc.subcore_barrier()
        _slab_copy(s, lambda b, n: pltpu.sync_copy(acc.at[pl.ds(b, n)],
                                                   out_ref.at[pl.ds(b, n)]))

    @pl.when(c == 0)
    def _():
        phase(g_a, out_a)

    @pl.when(c == 1)
    def _():
        phase(g_b, out_b)


_agg1_call = pl.kernel(
    _agg1_body,
    out_type=(jax.ShapeDtypeStruct((N, 128), _f32),
              jax.ShapeDtypeStruct((N, 128), _f32)),
    mesh=_MESH,
    compiler_params=_SC_PARAMS,
    scratch_types=[
        pltpu.VMEM_SHARED((N, 128), _f32),
        pltpu.VMEM((A1_ESUB,), _i32),
        pltpu.VMEM((A1_NCH, A1_CH), _i32),
        pltpu.VMEM((A1_CH,), _f32),
        pltpu.VMEM((A1_CH,), _f32),
        pltpu.VMEM((A1_CH, 128), _f32),
        pltpu.VMEM((A1_CH, 128), _f32),
        pltpu.SemaphoreType.DMA,
        pltpu.SemaphoreType.DMA,
        pltpu.SemaphoreType.DMA,
        pltpu.SemaphoreType.DMA,
    ],
)


A2_CH = 40
A2_NCH = E // (2 * NSUB) // A2_CH
A2_ESUB = E // (2 * NSUB)


def _agg2_body(g_hbm, zeros_hbm, row_hbm, col2d_hbm, ew_hbm, out_a, out_b,
               acc, rowf, col2d, ew0, ew1, g0, g1,
               sem0, sem1, seme0, seme1):
    c = lax.axis_index("c")
    s = lax.axis_index("s")
    tid = c * NSUB + s

    def init_from(src):
        _slab_copy(s, lambda b, n: pltpu.sync_copy(src.at[pl.ds(b, n)],
                                                   acc.at[pl.ds(b, n)]))

    @pl.when(c == 0)
    def _():
        init_from(g_hbm)

    @pl.when(c == 1)
    def _():
        init_from(zeros_hbm)

    pltpu.sync_copy(row_hbm.at[pl.ds(tid * A2_ESUB, A2_ESUB)], rowf)
    pltpu.sync_copy(col2d_hbm.at[tid], col2d)
    plsc.subcore_barrier()

    _agg_chunks(g_hbm, acc, rowf, col2d, ew_hbm, tid * A2_ESUB,
                ew0, ew1, g0, g1, sem0, sem1, seme0, seme1,
                A2_NCH, A2_CH)

    plsc.subcore_barrier()

    @pl.when(c == 0)
    def _():
        _slab_copy(s, lambda b, n: pltpu.sync_copy(acc.at[pl.ds(b, n)],
                                                   out_a.at[pl.ds(b, n)]))

    @pl.when(c == 1)
    def _():
        _slab_copy(s, lambda b, n: pltpu.sync_copy(acc.at[pl.ds(b, n)],
                                                   out_b.at[pl.ds(b, n)]))


_agg2_call = pl.kernel(
    _agg2_body,
    out_type=(jax.ShapeDtypeStruct((N, 128), _f32),
              jax.ShapeDtypeStruct((N, 128), _f32)),
    mesh=_MESH,
    compiler_params=_SC_PARAMS,
    scratch_types=[
        pltpu.VMEM_SHARED((N, 128), _f32),
        pltpu.VMEM((A2_ESUB,), _i32),
        pltpu.VMEM((A2_NCH, A2_CH), _i32),
        pltpu.VMEM((A2_CH,), _f32),
        pltpu.VMEM((A2_CH,), _f32),
        pltpu.VMEM((A2_CH, 128), _f32),
        pltpu.VMEM((A2_CH, 128), _f32),
        pltpu.SemaphoreType.DMA,
        pltpu.SemaphoreType.DMA,
        pltpu.SemaphoreType.DMA,
        pltpu.SemaphoreType.DMA,
    ],
)


def _mm1_body(x_ref, w_ref, h_ref):
    h_ref[...] = jnp.dot(x_ref[...], w_ref[...],
                         preferred_element_type=_f32)


_mm1_call = pl.pallas_call(
    _mm1_body,
    out_shape=jax.ShapeDtypeStruct((N, 256), _f32),
)


def _dis(da_ref, db_ref):
    deg = da_ref[:, 0:1] + db_ref[:, 0:1] + 1.0
    return lax.rsqrt(deg)


def _scale1_body(h_ref, da_ref, db_ref, ga_ref, gb_ref):
    g = h_ref[...] * _dis(da_ref, db_ref)
    ga_ref[...] = g[:, :128]
    gb_ref[...] = g[:, 128:]


_scale1_call = pl.pallas_call(
    _scale1_body,
    out_shape=(jax.ShapeDtypeStruct((N, 128), _f32),
               jax.ShapeDtypeStruct((N, 128), _f32)),
)


def _mm2_body(aa_ref, ab_ref, da_ref, db_ref, w2_ref, g2_ref):
    dis = _dis(da_ref, db_ref)
    out1 = jnp.concatenate([aa_ref[...], ab_ref[...]], axis=1) * dis
    out1 = jnp.maximum(out1, 0.0)
    g2_ref[...] = jnp.dot(out1, w2_ref[...], preferred_element_type=_f32) * dis


_mm2_call = pl.pallas_call(
    _mm2_body,
    out_shape=jax.ShapeDtypeStruct((N, 128), _f32),
)


def _fin_body(ba_ref, bb_ref, da_ref, db_ref, out_ref):
    dis = _dis(da_ref, db_ref)
    out_ref[...] = (ba_ref[...] + bb_ref[...]) * dis


_fin_call = pl.pallas_call(
    _fin_body,
    out_shape=jax.ShapeDtypeStruct((N, 128), _f32),
)


def kernel(x, edge_index, edge_weight, W1, W2):
    row = edge_index[0].astype(_i32)
    col = edge_index[1].astype(_i32)
    ew = edge_weight.astype(_f32)
    zeros128 = jnp.zeros((N, 128), _f32)
    col2d_a1 = col.reshape(NSUB, A1_NCH, A1_CH)
    col2d_a2 = col.reshape(2 * NSUB, A2_NCH, A2_CH)

    col3d_dg = col.reshape(2 * NSUB, DG_NCH, DG_CH)
    da, db = _deg_call(col3d_dg, ew, zeros128)
    h1 = _mm1_call(x, W1)
    ga, gb = _scale1_call(h1, da, db)
    aa, ab = _agg1_call(ga, gb, row, col2d_a1, ew)
    g2 = _mm2_call(aa, ab, da, db, W2)
    ba, bb = _agg2_call(g2, zeros128, row, col2d_a2, ew)
    return _fin_call(ba, bb, da, db)


# speedup vs baseline: 1.1163x; 1.1027x over previous
import dataclasses
import functools

import jax
import jax.numpy as jnp
from jax import lax
from jax.experimental import pallas as pl
from jax.experimental.pallas import tpu as pltpu
from jax.experimental.pallas import tpu_sc as plsc

N = 10000
E = 160000
NSUB = 16
SLAB = 624
LAST_SLAB = N - (NSUB - 1) * SLAB

_MESH = plsc.VectorSubcoreMesh(core_axis_name="c", subcore_axis_name="s")
_f32 = jnp.float32
_i32 = jnp.int32

_SC_PARAMS = pltpu.CompilerParams()
if "needs_layout_passes" in pltpu.CompilerParams.__dataclass_fields__:
    _SC_PARAMS = dataclasses.replace(_SC_PARAMS, needs_layout_passes=False)


def _slab_copy(s, do_copy):
    base = pl.multiple_of(s * SLAB, 8)

    @pl.when(s < NSUB - 1)
    def _():
        do_copy(base, SLAB)

    @pl.when(s == NSUB - 1)
    def _():
        do_copy((NSUB - 1) * SLAB, LAST_SLAB)


DG_CH = 40
DG_NCH = E // (2 * NSUB) // DG_CH
DG_ESUB = E // (2 * NSUB)


def _deg_body(col3d_hbm, ew_hbm, zeros_hbm, out_a, out_b,
              acc, col2d, ewf, v0, v1, sv0, sv1):
    c = lax.axis_index("c")
    s = lax.axis_index("s")
    tid = c * NSUB + s

    @pl.loop(0, DG_CH)
    def _zrow(r):
        for f in range(8):
            v0[r, pl.ds(f * 16, 16)] = jnp.zeros((16,), _f32)

    base = pl.multiple_of(s * SLAB, 8)
    for k in range(SLAB // DG_CH):
        pltpu.sync_copy(v0, acc.at[pl.ds(base + k * DG_CH, DG_CH)])
    pltpu.sync_copy(v0.at[pl.ds(0, SLAB - (SLAB // DG_CH) * DG_CH)],
                    acc.at[pl.ds(base + (SLAB // DG_CH) * DG_CH,
                                 SLAB - (SLAB // DG_CH) * DG_CH)])

    @pl.when(s == NSUB - 1)
    def _():
        extra = LAST_SLAB - SLAB
        pltpu.sync_copy(v0.at[pl.ds(0, extra)],
                        acc.at[pl.ds(N - extra, extra)])

    pltpu.sync_copy(col3d_hbm.at[tid], col2d)
    pltpu.sync_copy(ew_hbm.at[pl.ds(tid * DG_ESUB, DG_ESUB)], ewf)
    plsc.subcore_barrier()

    def build(m, vb):
        ebase = m * DG_CH

        @plsc.parallel_loop(0, DG_CH, unroll=4)
        def _edge(e):
            w = plsc.load_gather(ewf, [jnp.full((16,), ebase + e, _i32)])
            vb[e, pl.ds(0, 16)] = w

    def scat(m, vb, sem):
        pltpu.async_copy(vb, acc.at[col2d.at[m]], sem, add=True)

    def wait_scat(vb, sem):
        pltpu.make_async_copy(zeros_hbm.at[pl.ds(0, DG_CH)], vb, sem).wait()

    build(0, v0)
    scat(0, v0, sv0)

    @pl.loop(0, (DG_NCH - 1) // 2)
    def _pair(i):
        m0 = 2 * i + 1
        build(m0, v1)
        scat(m0, v1, sv1)
        wait_scat(v0, sv0)
        build(m0 + 1, v0)
        scat(m0 + 1, v0, sv0)
        wait_scat(v1, sv1)

    wait_scat(v0, sv0)
    plsc.subcore_barrier()

    @pl.when(c == 0)
    def _():
        _slab_copy(s, lambda b, n: pltpu.sync_copy(acc.at[pl.ds(b, n)],
                                                   out_a.at[pl.ds(b, n)]))

    @pl.when(c == 1)
    def _():
        _slab_copy(s, lambda b, n: pltpu.sync_copy(acc.at[pl.ds(b, n)],
                                                   out_b.at[pl.ds(b, n)]))


_deg_call = pl.kernel(
    _deg_body,
    out_type=(jax.ShapeDtypeStruct((N, 128), _f32),
              jax.ShapeDtypeStruct((N, 128), _f32)),
    mesh=_MESH,
    compiler_params=_SC_PARAMS,
    scratch_types=[
        pltpu.VMEM_SHARED((N, 128), _f32),
        pltpu.VMEM((DG_NCH, DG_CH), _i32),
        pltpu.VMEM((DG_ESUB,), _f32),
        pltpu.VMEM((DG_CH, 128), _f32),
        pltpu.VMEM((DG_CH, 128), _f32),
        pltpu.SemaphoreType.DMA,
        pltpu.SemaphoreType.DMA,
    ],
)


def _agg_chunks(g_ref, acc, rowf, ew_hbm, col_hbm, ew_base,
                gbufs, cbufs, ebufs, gsems, csems, esems, ssems,
                n_chunks, ch):
    assert n_chunks % 3 == 2

    def fetch(m, k):
        off = pl.multiple_of(ew_base + m * ch, 8)
        pltpu.async_copy(col_hbm.at[pl.ds(off, ch)], cbufs[k], csems[k])
        pltpu.async_copy(ew_hbm.at[pl.ds(off, ch)], ebufs[k], esems[k])
        roff = pl.multiple_of(m * ch, 8)
        pltpu.async_copy(g_ref.at[rowf.at[pl.ds(roff, ch)]], gbufs[k],
                         gsems[k])

    def wait_scat(k):
        pltpu.make_async_copy(g_ref.at[pl.ds(0, ch)], gbufs[k],
                              ssems[k]).wait()

    def mult_scat(k):
        gb, cb, eb = gbufs[k], cbufs[k], ebufs[k]
        pltpu.make_async_copy(g_ref.at[pl.ds(0, ch)], gb, gsems[k]).wait()
        pltpu.make_async_copy(ew_hbm.at[pl.ds(0, ch)], eb, esems[k]).wait()

        @plsc.parallel_loop(0, ch, unroll=4)
        def _edge(e):
            w = plsc.load_gather(eb, [jnp.full((16,), e, _i32)])
            for f in range(8):
                sl = pl.ds(f * 16, 16)
                gb[e, sl] = gb[e, sl] * w

        pltpu.make_async_copy(col_hbm.at[pl.ds(0, ch)], cb, csems[k]).wait()
        pltpu.async_copy(gb, acc.at[cb], ssems[k], add=True)

    fetch(0, 0)
    fetch(1, 1)

    @pl.loop(0, (n_chunks - 2) // 3)
    def _triple(j):
        m = j * 3
        for r in range(3):
            mult_scat(r)
            if r == 0:
                @pl.when(j > 0)
                def _():
                    wait_scat(2)
            else:
                wait_scat(r - 1)
            fetch(m + r + 2, (r + 2) % 3)

    mult_scat(0)
    wait_scat(2)
    mult_scat(1)
    wait_scat(0)
    wait_scat(1)


A1_CH = 80
A1_NCH = E // NSUB // A1_CH
A1_ESUB = E // NSUB


def _agg1_body(g_a, g_b, row_hbm, col_hbm, ew_hbm, out_a, out_b,
               acc, rowf, g0, g1, g2, c0, c1, c2, e0, e1, e2,
               sg0, sg1, sg2, sc0, sc1, sc2, se0, se1, se2,
               ss0, ss1, ss2):
    c = lax.axis_index("c")
    s = lax.axis_index("s")

    def phase(g_ref, out_ref):
        _slab_copy(s, lambda b, n: pltpu.sync_copy(g_ref.at[pl.ds(b, n)],
                                                   acc.at[pl.ds(b, n)]))
        pltpu.sync_copy(row_hbm.at[pl.ds(s * A1_ESUB, A1_ESUB)], rowf)
        plsc.subcore_barrier()

        _agg_chunks(g_ref, acc, rowf, ew_hbm, col_hbm, s * A1_ESUB,
                    (g0, g1, g2), (c0, c1, c2), (e0, e1, e2),
                    (sg0, sg1, sg2), (sc0, sc1, sc2), (se0, se1, se2),
                    (ss0, ss1, ss2), A1_NCH, A1_CH)

        plsc.subcore_barrier()
        _slab_copy(s, lambda b, n: pltpu.sync_copy(acc.at[pl.ds(b, n)],
                                                   out_ref.at[pl.ds(b, n)]))

    @pl.when(c == 0)
    def _():
        phase(g_a, out_a)

    @pl.when(c == 1)
    def _():
        phase(g_b, out_b)


_agg1_call = pl.kernel(
    _agg1_body,
    out_type=(jax.ShapeDtypeStruct((N, 128), _f32),
              jax.ShapeDtypeStruct((N, 128), _f32)),
    mesh=_MESH,
    compiler_params=_SC_PARAMS,
    scratch_types=(
        [pltpu.VMEM_SHARED((N, 128), _f32),
         pltpu.VMEM((A1_ESUB,), _i32)]
        + [pltpu.VMEM((A1_CH, 128), _f32)] * 3
        + [pltpu.VMEM((A1_CH,), _i32)] * 3
        + [pltpu.VMEM((A1_CH,), _f32)] * 3
        + [pltpu.SemaphoreType.DMA] * 12
    ),
)


A2_CH = 40
A2_NCH = E // (2 * NSUB) // A2_CH
A2_ESUB = E // (2 * NSUB)


def _agg2_body(g_hbm, zeros_hbm, row_hbm, col_hbm, ew_hbm, out_a, out_b,
               acc, rowf, g0, g1, g2, c0, c1, c2, e0, e1, e2,
               sg0, sg1, sg2, sc0, sc1, sc2, se0, se1, se2,
               ss0, ss1, ss2):
    c = lax.axis_index("c")
    s = lax.axis_index("s")
    tid = c * NSUB + s

    def init_from(src):
        _slab_copy(s, lambda b, n: pltpu.sync_copy(src.at[pl.ds(b, n)],
                                                   acc.at[pl.ds(b, n)]))

    @pl.when(c == 0)
    def _():
        init_from(g_hbm)

    @pl.when(c == 1)
    def _():
        init_from(zeros_hbm)

    pltpu.sync_copy(row_hbm.at[pl.ds(tid * A2_ESUB, A2_ESUB)], rowf)
    plsc.subcore_barrier()

    _agg_chunks(g_hbm, acc, rowf, ew_hbm, col_hbm, tid * A2_ESUB,
                (g0, g1, g2), (c0, c1, c2), (e0, e1, e2),
                (sg0, sg1, sg2), (sc0, sc1, sc2), (se0, se1, se2),
                (ss0, ss1, ss2), A2_NCH, A2_CH)

    plsc.subcore_barrier()

    @pl.when(c == 0)
    def _():
        _slab_copy(s, lambda b, n: pltpu.sync_copy(acc.at[pl.ds(b, n)],
                                                   out_a.at[pl.ds(b, n)]))

    @pl.when(c == 1)
    def _():
        _slab_copy(s, lambda b, n: pltpu.sync_copy(acc.at[pl.ds(b, n)],
                                                   out_b.at[pl.ds(b, n)]))


_agg2_call = pl.kernel(
    _agg2_body,
    out_type=(jax.ShapeDtypeStruct((N, 128), _f32),
              jax.ShapeDtypeStruct((N, 128), _f32)),
    mesh=_MESH,
    compiler_params=_SC_PARAMS,
    scratch_types=(
        [pltpu.VMEM_SHARED((N, 128), _f32),
         pltpu.VMEM((A2_ESUB,), _i32)]
        + [pltpu.VMEM((A2_CH, 128), _f32)] * 3
        + [pltpu.VMEM((A2_CH,), _i32)] * 3
        + [pltpu.VMEM((A2_CH,), _f32)] * 3
        + [pltpu.SemaphoreType.DMA] * 12
    ),
)


def _mm1_body(x_ref, w_ref, h_ref):
    h_ref[...] = jnp.dot(x_ref[...], w_ref[...],
                         preferred_element_type=_f32)


_mm1_call = pl.pallas_call(
    _mm1_body,
    out_shape=jax.ShapeDtypeStruct((N, 256), _f32),
)


def _dis(da_ref, db_ref):
    deg = da_ref[:, 0:1] + db_ref[:, 0:1] + 1.0
    return lax.rsqrt(deg)


def _scale1_body(h_ref, da_ref, db_ref, ga_ref, gb_ref):
    g = h_ref[...] * _dis(da_ref, db_ref)
    ga_ref[...] = g[:, :128]
    gb_ref[...] = g[:, 128:]


_scale1_call = pl.pallas_call(
    _scale1_body,
    out_shape=(jax.ShapeDtypeStruct((N, 128), _f32),
               jax.ShapeDtypeStruct((N, 128), _f32)),
)


def _mm2_body(aa_ref, ab_ref, da_ref, db_ref, w2_ref, g2_ref):
    dis = _dis(da_ref, db_ref)
    out1 = jnp.concatenate([aa_ref[...], ab_ref[...]], axis=1) * dis
    out1 = jnp.maximum(out1, 0.0)
    g2_ref[...] = jnp.dot(out1, w2_ref[...], preferred_element_type=_f32) * dis


_mm2_call = pl.pallas_call(
    _mm2_body,
    out_shape=jax.ShapeDtypeStruct((N, 128), _f32),
)


def _fin_body(ba_ref, bb_ref, da_ref, db_ref, out_ref):
    dis = _dis(da_ref, db_ref)
    out_ref[...] = (ba_ref[...] + bb_ref[...]) * dis


_fin_call = pl.pallas_call(
    _fin_body,
    out_shape=jax.ShapeDtypeStruct((N, 128), _f32),
)


def kernel(x, edge_index, edge_weight, W1, W2):
    row = edge_index[0].astype(_i32)
    col = edge_index[1].astype(_i32)
    ew = edge_weight.astype(_f32)
    zeros128 = jnp.zeros((N, 128), _f32)


    col3d_dg = col.reshape(2 * NSUB, DG_NCH, DG_CH)
    da, db = _deg_call(col3d_dg, ew, zeros128)
    h1 = _mm1_call(x, W1)
    ga, gb = _scale1_call(h1, da, db)
    aa, ab = _agg1_call(ga, gb, row, col, ew)
    g2 = _mm2_call(aa, ab, da, db, W2)
    ba, bb = _agg2_call(g2, zeros128, row, col, ew)
    return _fin_call(ba, bb, da, db)
